# Initial kernel scaffold; baseline (speedup 1.0000x reference)
import jax
import jax.numpy as jnp
from jax.experimental import pallas as pl


def kernel(x, adj_mat, W1, a1_l, a1_r, W2, a2_l, a2_r):
    raise NotImplementedError("write your pallas kernel here")


# speedup vs baseline: 1.6058x; 1.6058x over previous
import jax
import jax.numpy as jnp
from jax.experimental import pallas as pl

N = 2048
F_IN = 256
H1 = 8
D1 = 32
F_HID = 256
D2 = 32
BI = 256
NEG = -1e9
SLOPE = 0.2


def _proj1_kernel(x_ref, w_ref, al_ref, ar_ref, g_ref, el_ref, ert_ref):
    g = jnp.dot(x_ref[...], w_ref[...], preferred_element_type=jnp.float32)
    g_ref[...] = g
    el_ref[...] = jnp.dot(g, al_ref[...], preferred_element_type=jnp.float32)
    er = jnp.dot(g, ar_ref[...], preferred_element_type=jnp.float32)
    ert_ref[...] = er.T


def _masked_softmax_matmul(scores, adj, v):
    s = jnp.where(scores >= 0, scores, SLOPE * scores)
    s = jnp.where(adj, s, NEG)
    m = jnp.max(s, axis=1, keepdims=True)
    p = jnp.exp(s - m)
    denom = jnp.sum(p, axis=1, keepdims=True)
    o = jnp.dot(p, v, preferred_element_type=jnp.float32)
    return o / denom


def _layer1_kernel(g_ref, el_ref, ert_ref, adj_ref, w2_ref, a2l_ref, a2r_ref,
                   g2_ref, el2_ref, er2_ref):
    adj = adj_ref[...]
    g = g_ref[...]
    el = el_ref[...]
    outs = []
    for h in range(H1):
        scores = el[:, h:h + 1] + ert_ref[h:h + 1, :]
        outs.append(_masked_softmax_matmul(scores, adj, g[:, h * D1:(h + 1) * D1]))
    h1 = jnp.concatenate(outs, axis=1)
    h1 = jnp.where(h1 > 0, h1, jnp.exp(jnp.minimum(h1, 0.0)) - 1.0)
    g2 = jnp.dot(h1, w2_ref[...], preferred_element_type=jnp.float32)
    g2_ref[...] = g2
    el2_ref[...] = jnp.dot(g2, a2l_ref[...], preferred_element_type=jnp.float32)
    er2_ref[...] = jnp.dot(g2, a2r_ref[...], preferred_element_type=jnp.float32)


def _layer2_kernel(g2_ref, el2_ref, er2t_ref, adj_ref, out_ref):
    scores = el2_ref[...] + er2t_ref[...]
    out_ref[...] = _masked_softmax_matmul(scores, adj_ref[...], g2_ref[...])


def kernel(x, adj_mat, W1, a1_l, a1_r, W2, a2_l, a2_r):
    adj = adj_mat.reshape(N, N)
    eye = jnp.eye(H1, dtype=jnp.float32)
    Al = (eye[:, None, :] * a1_l[None, :, None]).reshape(F_HID, H1)
    Ar = (eye[:, None, :] * a1_r[None, :, None]).reshape(F_HID, H1)
    a2l = a2_l.reshape(D2, 1)
    a2r = a2_r.reshape(D2, 1)

    g1, el1, er1t = pl.pallas_call(
        _proj1_kernel,
        out_shape=[
            jax.ShapeDtypeStruct((N, F_HID), jnp.float32),
            jax.ShapeDtypeStruct((N, H1), jnp.float32),
            jax.ShapeDtypeStruct((H1, N), jnp.float32),
        ],
    )(x, W1, Al, Ar)

    nblk = N // BI
    g2, el2, er2 = pl.pallas_call(
        _layer1_kernel,
        grid=(nblk,),
        in_specs=[
            pl.BlockSpec((N, F_HID), lambda i: (0, 0)),
            pl.BlockSpec((BI, H1), lambda i: (i, 0)),
            pl.BlockSpec((H1, N), lambda i: (0, 0)),
            pl.BlockSpec((BI, N), lambda i: (i, 0)),
            pl.BlockSpec((F_HID, D2), lambda i: (0, 0)),
            pl.BlockSpec((D2, 1), lambda i: (0, 0)),
            pl.BlockSpec((D2, 1), lambda i: (0, 0)),
        ],
        out_specs=[
            pl.BlockSpec((BI, D2), lambda i: (i, 0)),
            pl.BlockSpec((BI, 1), lambda i: (i, 0)),
            pl.BlockSpec((BI, 1), lambda i: (i, 0)),
        ],
        out_shape=[
            jax.ShapeDtypeStruct((N, D2), jnp.float32),
            jax.ShapeDtypeStruct((N, 1), jnp.float32),
            jax.ShapeDtypeStruct((N, 1), jnp.float32),
        ],
    )(g1, el1, er1t, adj, W2, a2l, a2r)

    er2t = er2.reshape(1, N)
    out = pl.pallas_call(
        _layer2_kernel,
        grid=(nblk,),
        in_specs=[
            pl.BlockSpec((N, D2), lambda i: (0, 0)),
            pl.BlockSpec((BI, 1), lambda i: (i, 0)),
            pl.BlockSpec((1, N), lambda i: (0, 0)),
            pl.BlockSpec((BI, N), lambda i: (i, 0)),
        ],
        out_specs=pl.BlockSpec((BI, D2), lambda i: (i, 0)),
        out_shape=jax.ShapeDtypeStruct((N, D2), jnp.float32),
    )(g2, el2, er2t, adj)
    return out


# speedup vs baseline: 2.1299x; 1.3264x over previous
import jax
import jax.numpy as jnp
from jax.experimental import pallas as pl

N = 2048
F_IN = 256
H1 = 8
D1 = 32
F_HID = 256
D2 = 32
BI = 256
SLOPE = 0.2
LOG2E = 1.4426950408889634


def _proj1_kernel(x_ref, w_ref, al_ref, ar_ref, g_ref, el_ref, ert_ref):
    g = jnp.dot(x_ref[...], w_ref[...], preferred_element_type=jnp.float32)
    g_ref[...] = g
    el_ref[...] = jnp.dot(g, al_ref[...], preferred_element_type=jnp.float32) * LOG2E
    er = jnp.dot(g, ar_ref[...], preferred_element_type=jnp.float32) * LOG2E
    ert_ref[...] = er.T


def _attend(el_col, er_row, adj, vaug, vmean):
    ermax = jnp.max(er_row, axis=1, keepdims=True)
    mrow = el_col + ermax
    m = jnp.maximum(mrow, SLOPE * mrow)
    s = el_col + er_row
    l = jnp.maximum(s, SLOPE * s)
    pm = jnp.where(adj, jnp.exp2(l - m), 0.0)
    od = jnp.dot(pm, vaug, preferred_element_type=jnp.float32)
    o, denom = od[:, :-1], od[:, -1:]
    safe = denom > 0.0
    r = 1.0 / jnp.where(safe, denom, 1.0)
    return jnp.where(safe, o * r, vmean)


def _layer1_kernel(g_ref, el_ref, ert_ref, adj_ref, w2_ref, a2l_ref, a2r_ref,
                   g2_ref, el2_ref, er2_ref):
    adj = adj_ref[...]
    g = g_ref[...]
    el = el_ref[...]
    ones = jnp.ones((N, 1), dtype=jnp.float32)
    outs = []
    for h in range(H1):
        v = g[:, h * D1:(h + 1) * D1]
        vaug = jnp.concatenate([v, ones], axis=1)
        vmean = jnp.sum(v, axis=0, keepdims=True) * (1.0 / N)
        outs.append(_attend(el[:, h:h + 1], ert_ref[h:h + 1, :], adj, vaug, vmean))
    h1 = jnp.concatenate(outs, axis=1)
    h1 = jnp.where(h1 > 0, h1, jnp.exp(jnp.minimum(h1, 0.0)) - 1.0)
    g2 = jnp.dot(h1, w2_ref[...], preferred_element_type=jnp.float32)
    g2_ref[...] = g2
    el2_ref[...] = jnp.dot(g2, a2l_ref[...], preferred_element_type=jnp.float32) * LOG2E
    er2_ref[...] = jnp.dot(g2, a2r_ref[...], preferred_element_type=jnp.float32) * LOG2E


def _layer2_kernel(g2_ref, el2_ref, er2t_ref, adj_ref, out_ref):
    g2 = g2_ref[...]
    vaug = jnp.concatenate([g2, jnp.ones((N, 1), dtype=jnp.float32)], axis=1)
    vmean = jnp.sum(g2, axis=0, keepdims=True) * (1.0 / N)
    out_ref[...] = _attend(el2_ref[...], er2t_ref[...], adj_ref[...], vaug, vmean)


def kernel(x, adj_mat, W1, a1_l, a1_r, W2, a2_l, a2_r):
    adj = adj_mat.reshape(N, N)
    eye = jnp.eye(H1, dtype=jnp.float32)
    Al = (eye[:, None, :] * a1_l[None, :, None]).reshape(F_HID, H1)
    Ar = (eye[:, None, :] * a1_r[None, :, None]).reshape(F_HID, H1)
    a2l = a2_l.reshape(D2, 1)
    a2r = a2_r.reshape(D2, 1)

    g1, el1, er1t = pl.pallas_call(
        _proj1_kernel,
        out_shape=[
            jax.ShapeDtypeStruct((N, F_HID), jnp.float32),
            jax.ShapeDtypeStruct((N, H1), jnp.float32),
            jax.ShapeDtypeStruct((H1, N), jnp.float32),
        ],
    )(x, W1, Al, Ar)

    nblk = N // BI
    g2, el2, er2 = pl.pallas_call(
        _layer1_kernel,
        grid=(nblk,),
        in_specs=[
            pl.BlockSpec((N, F_HID), lambda i: (0, 0)),
            pl.BlockSpec((BI, H1), lambda i: (i, 0)),
            pl.BlockSpec((H1, N), lambda i: (0, 0)),
            pl.BlockSpec((BI, N), lambda i: (i, 0)),
            pl.BlockSpec((F_HID, D2), lambda i: (0, 0)),
            pl.BlockSpec((D2, 1), lambda i: (0, 0)),
            pl.BlockSpec((D2, 1), lambda i: (0, 0)),
        ],
        out_specs=[
            pl.BlockSpec((BI, D2), lambda i: (i, 0)),
            pl.BlockSpec((BI, 1), lambda i: (i, 0)),
            pl.BlockSpec((BI, 1), lambda i: (i, 0)),
        ],
        out_shape=[
            jax.ShapeDtypeStruct((N, D2), jnp.float32),
            jax.ShapeDtypeStruct((N, 1), jnp.float32),
            jax.ShapeDtypeStruct((N, 1), jnp.float32),
        ],
    )(g1, el1, er1t, adj, W2, a2l, a2r)

    er2t = er2.reshape(1, N)
    out = pl.pallas_call(
        _layer2_kernel,
        grid=(nblk,),
        in_specs=[
            pl.BlockSpec((N, D2), lambda i: (0, 0)),
            pl.BlockSpec((BI, 1), lambda i: (i, 0)),
            pl.BlockSpec((1, N), lambda i: (0, 0)),
            pl.BlockSpec((BI, N), lambda i: (i, 0)),
        ],
        out_specs=pl.BlockSpec((BI, D2), lambda i: (i, 0)),
        out_shape=jax.ShapeDtypeStruct((N, D2), jnp.float32),
    )(g2, el2, er2t, adj)
    return out
